# baseline (speedup 1.0000x reference)
import functools

import jax
import jax.numpy as jnp
from jax import lax
from jax.experimental import pallas as pl
from jax.experimental.pallas import tpu as pltpu
from jax.experimental.pallas import tpu_sc as plsc

_NC = 2
_NS = 16
_NW = _NC * _NS
_NB = 32
_L = 16


def _gather(idx, table4):
    (b,) = idx.shape
    _, g4, d2 = table4.shape
    d = d2 // 2
    bpw = b // _NW
    nbat = bpw // _NB
    mesh = plsc.VectorSubcoreMesh(core_axis_name="c", subcore_axis_name="s")

    @functools.partial(
        pl.kernel,
        mesh=mesh,
        out_type=jax.ShapeDtypeStruct((b, d), jnp.float32),
        scratch_types=[
            pltpu.VMEM((bpw + _L,), jnp.int32),
            pltpu.VMEM((bpw,), jnp.int32),
            pltpu.VMEM((2, _NB, g4, d2), jnp.float32),
            pltpu.VMEM((2, _NB, d), jnp.float32),
            pltpu.SemaphoreType.DMA,
            pltpu.SemaphoreType.DMA,
            pltpu.SemaphoreType.DMA,
        ],
    )
    def k(idx_hbm, tbl_hbm, out_hbm, idx_v, tid_v, tiles_v, rows_v,
          sem_g0, sem_g1, sem_o):
        wid = lax.axis_index("s") * _NC + lax.axis_index("c")
        base = wid * bpw
        pltpu.sync_copy(idx_hbm.at[pl.ds(base, bpw)], idx_v.at[pl.ds(0, bpw)])
        for t in range(bpw // _L):
            v = idx_v[pl.ds(t * _L, _L)]
            tid_v[pl.ds(t * _L, _L)] = lax.shift_right_logical(v, 3)
        sems = (sem_g0, sem_g1)

        def gather_batch(g, buf):
            return pltpu.async_copy(
                tbl_hbm.at[tid_v.at[pl.ds(g * _NB, _NB)]], tiles_v.at[buf],
                sems[g % 2])

        gather_batch(0, 0)
        for g in range(nbat):
            p = g % 2
            pltpu.make_async_copy(
                tbl_hbm.at[tid_v.at[pl.ds(g * _NB, _NB)]], tiles_v.at[p],
                sems[g % 2]).wait()
            if g + 1 < nbat:
                gather_batch(g + 1, 1 - p)
            if g >= 2:
                pltpu.make_async_copy(
                    rows_v.at[p], out_hbm.at[pl.ds(base + (g - 2) * _NB, _NB)],
                    sem_o).wait()

            def extract(i, _):
                r = jnp.bitwise_and(idx_v[pl.ds(g * _NB + i, _L)][0], 7)
                rhi = lax.shift_right_logical(r, 1)
                rlo = jnp.bitwise_and(r, 1) * d
                for c in range(d // _L):
                    rows_v[p, i, pl.ds(c * _L, _L)] = (
                        tiles_v[p, i, rhi, pl.ds(rlo + c * _L, _L)])
                return ()

            lax.fori_loop(0, _NB, extract, ())
            pltpu.async_copy(
                rows_v.at[p], out_hbm.at[pl.ds(base + g * _NB, _NB)], sem_o)
        for g in range(nbat - 2, nbat):
            pltpu.make_async_copy(
                rows_v.at[g % 2], out_hbm.at[pl.ds(base + g * _NB, _NB)],
                sem_o).wait()

    return k(idx, table4)


def kernel(input, use_blank, has_blank, table):
    v, d = table.shape
    table4 = table.reshape(v // 8, 4, 2 * d)
    return _gather(input.astype(jnp.int32), table4)


# speedup vs baseline: 2.5076x; 2.5076x over previous
import functools

import jax
import jax.numpy as jnp
from jax import lax
from jax.experimental import pallas as pl
from jax.experimental.pallas import tpu as pltpu
from jax.experimental.pallas import tpu_sc as plsc

_NC = 2
_NS = 16
_NW = _NC * _NS
_NB = 32
_L = 16


def _gather(idx, table3):
    (b,) = idx.shape
    _, r8, d = table3.shape
    bpw = b // _NW
    nbat = bpw // _NB
    mesh = plsc.VectorSubcoreMesh(core_axis_name="c", subcore_axis_name="s")

    @functools.partial(
        pl.kernel,
        mesh=mesh,
        out_type=jax.ShapeDtypeStruct((b, d), table3.dtype),
        scratch_types=[
            pltpu.VMEM((bpw + _L,), jnp.int32),
            pltpu.VMEM((3, _NB, r8, d), table3.dtype),
            pltpu.VMEM((2, _NB, d), table3.dtype),
            pltpu.SemaphoreType.DMA,
            pltpu.SemaphoreType.DMA,
            pltpu.SemaphoreType.DMA,
            pltpu.SemaphoreType.DMA,
        ],
    )
    def k(idx_hbm, tbl_hbm, out_hbm, idx_v, tiles_v, rows_v,
          sem_g0, sem_g1, sem_g2, sem_o):
        wid = lax.axis_index("s") * _NC + lax.axis_index("c")
        base = wid * bpw
        pltpu.sync_copy(idx_hbm.at[pl.ds(base, bpw)], idx_v.at[pl.ds(0, bpw)])
        sems = (sem_g0, sem_g1, sem_g2)

        def issue_batch(g, buf):
            sem = sems[g % 3]

            def issue(i, _):
                vv = idx_v[pl.ds(g * _NB + i, _L)]
                t = lax.shift_right_logical(vv[0], 3)
                pltpu.async_copy(tbl_hbm.at[t], tiles_v.at[buf, i], sem)
                return ()

            lax.fori_loop(0, _NB, issue, ())

        def wait_batch(g, buf):
            pltpu.make_async_copy(
                tbl_hbm.at[pl.ds(0, _NB)], tiles_v.at[buf], sems[g % 3]).wait()

        issue_batch(0, 0)
        issue_batch(1, 1)
        for g in range(nbat):
            p = g % 3
            po = g % 2
            wait_batch(g, p)
            if g + 2 < nbat:
                issue_batch(g + 2, (g + 2) % 3)
            if g >= 2:
                pltpu.make_async_copy(
                    rows_v.at[po],
                    out_hbm.at[pl.ds(base + (g - 2) * _NB, _NB)],
                    sem_o).wait()

            def extract(i, _):
                vv = idx_v[pl.ds(g * _NB + i, _L)]
                r = jnp.bitwise_and(vv[0], 7)
                for c in range(d // _L):
                    rows_v[po, i, pl.ds(c * _L, _L)] = (
                        tiles_v[p, i, r, pl.ds(c * _L, _L)])
                return ()

            lax.fori_loop(0, _NB, extract, ())
            pltpu.async_copy(
                rows_v.at[po], out_hbm.at[pl.ds(base + g * _NB, _NB)], sem_o)
        for g in range(nbat - 2, nbat):
            pltpu.make_async_copy(
                rows_v.at[g % 2], out_hbm.at[pl.ds(base + g * _NB, _NB)],
                sem_o).wait()

    return k(idx, table3)


def kernel(input, use_blank, has_blank, table):
    v, d = table.shape
    table3 = table.reshape(v // 8, 8, d)
    return _gather(input.astype(jnp.int32), table3)
